# baseline (speedup 1.0000x reference)
import functools

import jax
import jax.numpy as jnp
from jax.experimental import pallas as pl

N = 2048
BM = 256
NB = N // BM
CUTOFF2 = 1.5 * 1.5


def _w_block(jb, Ar, Bc):
    rows = slice(jb * BM, (jb + 1) * BM)
    d2 = jax.lax.dot_general(Ar[rows, :], Bc, (((1,), (0,)), ((), ())),
                             precision=jax.lax.Precision.HIGHEST,
                             preferred_element_type=jnp.float32)
    row_ids = jax.lax.broadcasted_iota(jnp.int32, (BM, N), 0) + jb * BM
    col_ids = jax.lax.broadcasted_iota(jnp.int32, (BM, N), 1)
    mask = (d2 < CUTOFF2) & (row_ids != col_ids)
    W = jnp.where(mask, jax.lax.rsqrt(jnp.maximum(d2, 0.0) + 1e-12), 0.0)
    deg = jnp.sum(jnp.where(mask, 1.0, 0.0), axis=1, keepdims=True)
    invdeg = 1.0 / jnp.maximum(deg, 1.0)
    has = (deg > 0.0).astype(jnp.float32)
    return W, invdeg, has


def _body(ar_ref, bc_ref, ab_ref, wo_ref, bo_ref, out_ref):
    Ar = ar_ref[:]
    Bc = bc_ref[:]
    a = ab_ref[0:1, :]
    b = ab_ref[1:2, :]
    X4 = Ar[:, 0:4]

    uc_blocks = []
    aux = []
    for jb in range(NB):
        W, invdeg, has = _w_block(jb, Ar, Bc)
        Pr = X4[jb * BM:(jb + 1) * BM, 0:3]
        A = jnp.dot(W, X4, preferred_element_type=jnp.float32)
        U = (Pr * A[:, 3:4] - A[:, 0:3]) * invdeg
        c = jnp.sum(U * Pr, axis=1, keepdims=True)
        uc_blocks.append(jnp.concatenate([U, c], axis=1))
        aux.append((invdeg, has, Pr))
    UC = jnp.concatenate(uc_blocks, axis=0)

    acc = jnp.zeros((1, 10), dtype=jnp.float32)
    for jb in range(NB):
        W, invdeg, has = _w_block(jb, Ar, Bc)
        Pr = aux[jb][2]
        B = jnp.dot(W, UC, preferred_element_type=jnp.float32)
        t = (jnp.sum(Pr * B[:, 0:3], axis=1, keepdims=True)
             - B[:, 3:4]) * invdeg
        node2 = jax.nn.gelu(has * a + t * b)
        acc = acc + jnp.sum(node2, axis=0, keepdims=True)

    pooled = acc * (1.0 / N)
    logits = jnp.dot(pooled, wo_ref[:],
                     preferred_element_type=jnp.float32) + bo_ref[:]
    out_ref[:] = jax.nn.softmax(logits, axis=-1)


@functools.partial(jax.jit, static_argnames=())
def kernel(positions, w1_s, w1_v, w2_ss, w2_vv, W_out, b_out):
    sq = jnp.sum(positions * positions, axis=1)
    ones = jnp.ones((N, 1), jnp.float32)
    zeros = jnp.zeros((N, 3), jnp.float32)
    Ar = jnp.concatenate([positions, ones, sq[:, None], zeros], axis=1)
    Bc = jnp.concatenate([-2.0 * positions.T, sq[None, :],
                          jnp.ones((1, N), jnp.float32),
                          jnp.zeros((3, N), jnp.float32)], axis=0)
    ab = jnp.stack([jax.nn.gelu(w1_s) @ w2_ss, w1_v @ w2_vv], axis=0)
    out = pl.pallas_call(
        _body,
        out_shape=jax.ShapeDtypeStruct((1, 10), jnp.float32),
    )(Ar, Bc, ab, W_out, b_out[None, :])
    return out[0]


# speedup vs baseline: 1.3805x; 1.3805x over previous
import functools

import jax
import jax.numpy as jnp
from jax.experimental import pallas as pl

N = 2048
BM = 256
NB = N // BM
CUTOFF2 = 1.5 * 1.5


def _w_block(jb, P, sq, Pt):
    rows = slice(jb * BM, (jb + 1) * BM)
    Pr = P[rows, :]
    sqr = sq[rows, :]
    sqc = Pt[3:4, :]
    d2 = sqr + sqc - 2.0 * (
        Pr[:, 0:1] * Pt[0:1, :]
        + Pr[:, 1:2] * Pt[1:2, :]
        + Pr[:, 2:3] * Pt[2:3, :]
    )
    row_ids = jax.lax.broadcasted_iota(jnp.int32, (BM, N), 0) + jb * BM
    col_ids = jax.lax.broadcasted_iota(jnp.int32, (BM, N), 1)
    mask = (d2 < CUTOFF2) & (row_ids != col_ids)
    W = jnp.where(mask, jax.lax.rsqrt(jnp.maximum(d2, 0.0) + 1e-12), 0.0)
    deg = jnp.sum(jnp.where(mask, 1.0, 0.0), axis=1, keepdims=True)
    invdeg = 1.0 / jnp.maximum(deg, 1.0)
    has = (deg > 0.0).astype(jnp.float32)
    return W, Pr, invdeg, has


def _body(p_ref, ab_ref, wo_ref, bo_ref, out_ref):
    P = p_ref[:]
    sq = jnp.sum(P * P, axis=1, keepdims=True)
    Pt = jnp.transpose(jnp.concatenate([P, sq], axis=1))
    X4 = jnp.concatenate([P, jnp.ones((N, 1), jnp.float32)], axis=1)
    a = ab_ref[0:1, :]
    b = ab_ref[1:2, :]

    uc_blocks = []
    aux = []
    for jb in range(NB):
        W, Pr, invdeg, has = _w_block(jb, P, sq, Pt)
        A = jnp.dot(W, X4, preferred_element_type=jnp.float32)
        U = (Pr * A[:, 3:4] - A[:, 0:3]) * invdeg
        c = jnp.sum(U * Pr, axis=1, keepdims=True)
        uc_blocks.append(jnp.concatenate([U, c], axis=1))
        aux.append((invdeg, has))
    UC = jnp.concatenate(uc_blocks, axis=0)

    acc = jnp.zeros((1, 10), dtype=jnp.float32)
    for jb in range(NB):
        W, Pr, invdeg, has = _w_block(jb, P, sq, Pt)
        B = jnp.dot(W, UC, preferred_element_type=jnp.float32)
        t = (jnp.sum(Pr * B[:, 0:3], axis=1, keepdims=True)
             - B[:, 3:4]) * invdeg
        node2 = jax.nn.gelu(has * a + t * b)
        acc = acc + jnp.sum(node2, axis=0, keepdims=True)

    pooled = acc * (1.0 / N)
    logits = jnp.dot(pooled, wo_ref[:],
                     preferred_element_type=jnp.float32) + bo_ref[:]
    out_ref[:] = jax.nn.softmax(logits, axis=-1)


@functools.partial(jax.jit, static_argnames=())
def kernel(positions, w1_s, w1_v, w2_ss, w2_vv, W_out, b_out):
    ab = jnp.stack([jax.nn.gelu(w1_s) @ w2_ss, w1_v @ w2_vv], axis=0)
    out = pl.pallas_call(
        _body,
        out_shape=jax.ShapeDtypeStruct((1, 10), jnp.float32),
    )(positions, ab, W_out, b_out[None, :])
    return out[0]


# speedup vs baseline: 1.4551x; 1.0540x over previous
import functools

import jax
import jax.numpy as jnp
from jax.experimental import pallas as pl

N = 2048
BM = 256
NB = N // BM
CUTOFF2 = 1.5 * 1.5


def _w_block(jb, P, sq, Ptm, eyeb, with_deg):
    rows = slice(jb * BM, (jb + 1) * BM)
    Pr = P[rows, :]
    sqr = sq[rows, :]
    d2 = (sqr + Ptm[3:4, :]) + (
        Pr[:, 0:1] * Ptm[0:1, :]
        + Pr[:, 1:2] * Ptm[1:2, :]
        + Pr[:, 2:3] * Ptm[2:3, :]
    )
    mask = d2 < CUTOFF2
    W0 = jnp.where(mask, jax.lax.rsqrt(jnp.maximum(d2, 1e-12)), 0.0)
    s = jb * BM
    Wd = jnp.where(eyeb, 0.0, W0[:, s:s + BM])
    parts = ([W0[:, :s]] if s > 0 else []) + [Wd] + \
        ([W0[:, s + BM:]] if s + BM < N else [])
    W = jnp.concatenate(parts, axis=1)
    deg = (jnp.sum(jnp.where(mask, 1.0, 0.0), axis=1, keepdims=True)
           if with_deg else None)
    return W, deg


def _body(p_ref, ab_ref, wo_ref, bo_ref, out_ref):
    P = p_ref[:]
    sq = jnp.sum(P * P, axis=1, keepdims=True)
    Ptm = jnp.transpose(
        jnp.concatenate([-2.0 * P, sq], axis=1))
    X4 = jnp.concatenate([P, jnp.ones((N, 1), jnp.float32)], axis=1)
    eyeb = (jax.lax.broadcasted_iota(jnp.int32, (BM, BM), 0)
            == jax.lax.broadcasted_iota(jnp.int32, (BM, BM), 1))
    a = ab_ref[0:1, :]
    b = ab_ref[1:2, :]

    uc_blocks = []
    aux = []
    for jb in range(NB):
        W, deg1 = _w_block(jb, P, sq, Ptm, eyeb, True)
        deg = deg1 - 1.0
        invdeg = 1.0 / jnp.maximum(deg, 1.0)
        has = (deg > 0.0).astype(jnp.float32)
        Pr = P[jb * BM:(jb + 1) * BM, :]
        A = jnp.dot(W, X4, preferred_element_type=jnp.float32)
        U = (Pr * A[:, 3:4] - A[:, 0:3]) * invdeg
        c = jnp.sum(U * Pr, axis=1, keepdims=True)
        uc_blocks.append(jnp.concatenate([U, c], axis=1))
        aux.append((invdeg, has, Pr))
    UC = jnp.concatenate(uc_blocks, axis=0)

    acc = jnp.zeros((1, 10), dtype=jnp.float32)
    for jb in range(NB):
        W, _ = _w_block(jb, P, sq, Ptm, eyeb, False)
        invdeg, has, Pr = aux[jb]
        B = jnp.dot(W, UC, preferred_element_type=jnp.float32)
        t = (jnp.sum(Pr * B[:, 0:3], axis=1, keepdims=True)
             - B[:, 3:4]) * invdeg
        node2 = jax.nn.gelu(has * a + t * b)
        acc = acc + jnp.sum(node2, axis=0, keepdims=True)

    pooled = acc * (1.0 / N)
    logits = jnp.dot(pooled, wo_ref[:],
                     preferred_element_type=jnp.float32) + bo_ref[:]
    out_ref[:] = jax.nn.softmax(logits, axis=-1)


@functools.partial(jax.jit, static_argnames=())
def kernel(positions, w1_s, w1_v, w2_ss, w2_vv, W_out, b_out):
    ab = jnp.stack([jax.nn.gelu(w1_s) @ w2_ss, w1_v @ w2_vv], axis=0)
    out = pl.pallas_call(
        _body,
        out_shape=jax.ShapeDtypeStruct((1, 10), jnp.float32),
    )(positions, ab, W_out, b_out[None, :])
    return out[0]


# speedup vs baseline: 1.4664x; 1.0078x over previous
import functools

import jax
import jax.numpy as jnp
from jax.experimental import pallas as pl

N = 2048
BM = 512
NB = N // BM
CUTOFF2 = 1.5 * 1.5


def _w_block(jb, P, sq, Ptm, eyeb, with_deg):
    rows = slice(jb * BM, (jb + 1) * BM)
    Pr = P[rows, :]
    sqr = sq[rows, :]
    d2 = (sqr + Ptm[3:4, :]) + (
        Pr[:, 0:1] * Ptm[0:1, :]
        + Pr[:, 1:2] * Ptm[1:2, :]
        + Pr[:, 2:3] * Ptm[2:3, :]
    )
    mask = d2 < CUTOFF2
    W0 = jnp.where(mask, jax.lax.rsqrt(jnp.maximum(d2, 1e-12)), 0.0)
    s = jb * BM
    Wd = jnp.where(eyeb, 0.0, W0[:, s:s + BM])
    parts = ([W0[:, :s]] if s > 0 else []) + [Wd] + \
        ([W0[:, s + BM:]] if s + BM < N else [])
    W = jnp.concatenate(parts, axis=1)
    deg = (jnp.sum(jnp.where(mask, 1.0, 0.0), axis=1, keepdims=True)
           if with_deg else None)
    return W, deg


def _body(p_ref, ab_ref, wo_ref, bo_ref, out_ref):
    P = p_ref[:]
    sq = jnp.sum(P * P, axis=1, keepdims=True)
    Ptm = jnp.transpose(
        jnp.concatenate([-2.0 * P, sq], axis=1))
    X4 = jnp.concatenate([P, jnp.ones((N, 1), jnp.float32)], axis=1)
    eyeb = (jax.lax.broadcasted_iota(jnp.int32, (BM, BM), 0)
            == jax.lax.broadcasted_iota(jnp.int32, (BM, BM), 1))
    a = ab_ref[0:1, :]
    b = ab_ref[1:2, :]

    uc_blocks = []
    aux = []
    for jb in range(NB):
        W, deg1 = _w_block(jb, P, sq, Ptm, eyeb, True)
        deg = deg1 - 1.0
        invdeg = 1.0 / jnp.maximum(deg, 1.0)
        has = (deg > 0.0).astype(jnp.float32)
        Pr = P[jb * BM:(jb + 1) * BM, :]
        A = jnp.dot(W, X4, preferred_element_type=jnp.float32)
        U = (Pr * A[:, 3:4] - A[:, 0:3]) * invdeg
        c = jnp.sum(U * Pr, axis=1, keepdims=True)
        uc_blocks.append(jnp.concatenate([U, c], axis=1))
        aux.append((invdeg, has, Pr))
    UC = jnp.concatenate(uc_blocks, axis=0)

    acc = jnp.zeros((1, 10), dtype=jnp.float32)
    for jb in range(NB):
        W, _ = _w_block(jb, P, sq, Ptm, eyeb, False)
        invdeg, has, Pr = aux[jb]
        B = jnp.dot(W, UC, preferred_element_type=jnp.float32)
        t = (jnp.sum(Pr * B[:, 0:3], axis=1, keepdims=True)
             - B[:, 3:4]) * invdeg
        node2 = jax.nn.gelu(has * a + t * b)
        acc = acc + jnp.sum(node2, axis=0, keepdims=True)

    pooled = acc * (1.0 / N)
    logits = jnp.dot(pooled, wo_ref[:],
                     preferred_element_type=jnp.float32) + bo_ref[:]
    out_ref[:] = jax.nn.softmax(logits, axis=-1)


@functools.partial(jax.jit, static_argnames=())
def kernel(positions, w1_s, w1_v, w2_ss, w2_vv, W_out, b_out):
    ab = jnp.stack([jax.nn.gelu(w1_s) @ w2_ss, w1_v @ w2_vv], axis=0)
    out = pl.pallas_call(
        _body,
        out_shape=jax.ShapeDtypeStruct((1, 10), jnp.float32),
    )(positions, ab, W_out, b_out[None, :])
    return out[0]
